# baseline (speedup 1.0000x reference)
import functools

import jax
import jax.numpy as jnp
from jax import lax
from jax.experimental import pallas as pl
from jax.experimental.pallas import tpu as pltpu
from jax.experimental.pallas import tpu_sc as plsc

N = 10000
NP = 10240
E = 160000
DF = 128
HID = 64
NCLS = 40

NC, NS = 2, 16
HALF = NP // 2
ET = E // NS
BLK = 2000
NBLK = ET // BLK
SLAB = NP * NP // (NC * NS)
ZCH = 16384
IDXROWS = 32


def _sc_build_adj(edge_flat):
    mesh = plsc.VectorSubcoreMesh(
        core_axis_name="c", subcore_axis_name="s", num_cores=NC, num_subcores=NS)

    @functools.partial(
        pl.kernel,
        out_type=jax.ShapeDtypeStruct((NP * NP,), jnp.float32),
        mesh=mesh,
        scratch_types=[
            pltpu.VMEM((ZCH,), jnp.float32),
            pltpu.VMEM((BLK,), jnp.int32),
            pltpu.VMEM((BLK,), jnp.int32),
            pltpu.VMEM((IDXROWS, 128), jnp.int32),
            pltpu.VMEM((128,), jnp.float32),
            pltpu.SemaphoreType.DMA,
        ],
    )
    def build(edge_hbm, out_hbm, zbuf, sbuf, dbuf, idxbuf, ones_v, sem):
        c = lax.axis_index("c")
        s = lax.axis_index("s")
        lo = c * HALF
        hi = lo + HALF
        dummy = lo * NP + lo

        def zb(i, _):
            zbuf[pl.ds(i * 16, 16)] = jnp.zeros((16,), jnp.float32)
            return 0
        lax.fori_loop(0, ZCH // 16, zb, 0)
        for j in range(8):
            ones_v[pl.ds(j * 16, 16)] = jnp.ones((16,), jnp.float32)
        for j in range((IDXROWS * 128 - 2 * BLK) // 16):
            p = 2 * BLK + j * 16
            idxbuf[p // 128, pl.ds(p % 128, 16)] = jnp.zeros((16,), jnp.int32) + dummy

        zbase = (c * NS + s) * SLAB

        def zero_step(i, _):
            pltpu.sync_copy(zbuf, out_hbm.at[pl.ds(zbase + i * ZCH, ZCH)])
            return 0
        lax.fori_loop(0, SLAB // ZCH, zero_step, 0)

        plsc.subcore_barrier()

        for b in range(NBLK):
            ebase = s * ET + b * BLK
            pltpu.sync_copy(edge_hbm.at[pl.ds(ebase, BLK)], sbuf)
            pltpu.sync_copy(edge_hbm.at[pl.ds(E + ebase, BLK)], dbuf)

            def fill(i, _):
                sv = sbuf[pl.ds(i * 16, 16)]
                dv = dbuf[pl.ds(i * 16, 16)]
                own1 = (dv >= lo) & (dv < hi)
                idx1 = jnp.where(own1, dv * NP + sv, dummy)
                own2 = (sv >= lo) & (sv < hi)
                idx2 = jnp.where(own2, sv * NP + dv, dummy)
                p1 = i * 16
                idxbuf[p1 // 128, pl.ds(p1 % 128, 16)] = idx1
                p2 = p1 + BLK
                idxbuf[p2 // 128, pl.ds(p2 % 128, 16)] = idx2
                return 0
            lax.fori_loop(0, BLK // 16, fill, 0)

            for j in range(IDXROWS):
                pltpu.async_copy(ones_v, out_hbm.at[idxbuf.at[j]], sem).wait()

    return build(edge_flat)


def _prep(counts2d):
    RA = 256

    def body(c_ref, ab_ref, d_ref):
        i = pl.program_id(0)
        cts = c_ref[...]
        rows = i * RA + lax.broadcasted_iota(jnp.int32, (RA, NP), 0)
        cols = lax.broadcasted_iota(jnp.int32, (RA, NP), 1)
        ab = (cts > 0.0) | ((rows == cols) & (rows < N))
        abf = ab.astype(jnp.float32)
        ab_ref[...] = abf.astype(jnp.bfloat16)
        d_ref[...] = jnp.sum(abf, axis=1, keepdims=True)

    return pl.pallas_call(
        body,
        grid=(NP // RA,),
        in_specs=[pl.BlockSpec((RA, NP), lambda i: (i, 0))],
        out_specs=[pl.BlockSpec((RA, NP), lambda i: (i, 0)),
                   pl.BlockSpec((RA, 1), lambda i: (i, 0))],
        out_shape=[jax.ShapeDtypeStruct((NP, NP), jnp.bfloat16),
                   jax.ShapeDtypeStruct((NP, 1), jnp.float32)],
    )(counts2d)


def _two_hop(ahat, dinv_row):
    RB, NB, KB = 512, 2048, 512
    RT, NT, KT = NP // RB, NP // NB, NP // KB

    def body(lhs_ref, rhs_ref, dinv_ref, afull_ref, m2_ref, deg2_ref, acc_ref):
        r, n, k = pl.program_id(0), pl.program_id(1), pl.program_id(2)
        lhs = lhs_ref[:, pl.ds(k * KB, KB)]
        scale = dinv_ref[...].astype(jnp.bfloat16)
        part = jnp.dot(lhs * scale, rhs_ref[...],
                       preferred_element_type=jnp.float32)

        @pl.when(k == 0)
        def _():
            acc_ref[...] = part

        @pl.when(k != 0)
        def _():
            acc_ref[...] += part

        @pl.when(k == KT - 1)
        def _():
            w = acc_ref[...]
            af = afull_ref[...].astype(jnp.float32)
            rows = r * RB + lax.broadcasted_iota(jnp.int32, (RB, NB), 0)
            cols = n * NB + lax.broadcasted_iota(jnp.int32, (RB, NB), 1)
            m2 = (w > af) & (rows != cols)
            m2f = m2.astype(jnp.float32)
            m2_ref[...] = m2f.astype(jnp.int8)
            part_deg = jnp.sum(m2f, axis=1, keepdims=True)

            @pl.when(n == 0)
            def _():
                deg2_ref[...] = part_deg

            @pl.when(n != 0)
            def _():
                deg2_ref[...] += part_deg

    return pl.pallas_call(
        body,
        grid=(RT, NT, KT),
        in_specs=[
            pl.BlockSpec((RB, NP), lambda r, n, k: (r, 0)),
            pl.BlockSpec((KB, NB), lambda r, n, k: (k, n)),
            pl.BlockSpec((1, KB), lambda r, n, k: (0, k)),
            pl.BlockSpec((RB, NB), lambda r, n, k: (r, n)),
        ],
        out_specs=[pl.BlockSpec((RB, NB), lambda r, n, k: (r, n)),
                   pl.BlockSpec((RB, 1), lambda r, n, k: (r, 0))],
        out_shape=[jax.ShapeDtypeStruct((NP, NP), jnp.int8),
                   jax.ShapeDtypeStruct((NP, 1), jnp.float32)],
        scratch_shapes=[pltpu.VMEM((RB, NB), jnp.float32)],
    )(ahat, ahat, dinv_row, ahat)


def _spmm_round(ahat, m2, r, d1c, d2c, d1r, d2r):
    F = r.shape[1]
    RB, KB = 512, 512
    RT, KT = NP // RB, NP // KB

    def body(a_ref, m_ref, rk_ref, d1k_ref, d2k_ref, rd_ref, d1cr_ref,
             d1rr_ref, d2rr_ref, out_ref, acc1_ref, acc2_ref):
        k = pl.program_id(1)
        rk = rk_ref[...]
        v1 = (rk * d1k_ref[...]).astype(jnp.bfloat16)
        v2 = (rk * d2k_ref[...]).astype(jnp.bfloat16)
        p1 = jnp.dot(a_ref[...], v1, preferred_element_type=jnp.float32)
        p2 = jnp.dot(m_ref[...].astype(jnp.bfloat16), v2,
                     preferred_element_type=jnp.float32)

        @pl.when(k == 0)
        def _():
            acc1_ref[...] = p1
            acc2_ref[...] = p2

        @pl.when(k != 0)
        def _():
            acc1_ref[...] += p1
            acc2_ref[...] += p2

        @pl.when(k == KT - 1)
        def _():
            h1 = (acc1_ref[...] - rd_ref[...] * d1cr_ref[...]) * d1rr_ref[...]
            h2 = acc2_ref[...] * d2rr_ref[...]
            out_ref[...] = jnp.concatenate(
                [jnp.maximum(h1, 0.0), jnp.maximum(h2, 0.0)], axis=1)

    return pl.pallas_call(
        body,
        grid=(RT, KT),
        in_specs=[
            pl.BlockSpec((RB, KB), lambda i, k: (i, k)),
            pl.BlockSpec((RB, KB), lambda i, k: (i, k)),
            pl.BlockSpec((KB, F), lambda i, k: (k, 0)),
            pl.BlockSpec((KB, 1), lambda i, k: (k, 0)),
            pl.BlockSpec((KB, 1), lambda i, k: (k, 0)),
            pl.BlockSpec((RB, F), lambda i, k: (i, 0)),
            pl.BlockSpec((RB, 1), lambda i, k: (i, 0)),
            pl.BlockSpec((RB, 1), lambda i, k: (i, 0)),
            pl.BlockSpec((RB, 1), lambda i, k: (i, 0)),
        ],
        out_specs=pl.BlockSpec((RB, 2 * F), lambda i, k: (i, 0)),
        out_shape=jax.ShapeDtypeStruct((NP, 2 * F), jnp.float32),
        scratch_shapes=[pltpu.VMEM((RB, F), jnp.float32),
                        pltpu.VMEM((RB, F), jnp.float32)],
    )(ahat, m2, r, d1c, d2c, r, d1c, d1r, d2r)


def _embed(xp, w):
    RB = 1024

    def body(x_ref, w_ref, o_ref):
        o_ref[...] = jnp.maximum(
            jnp.dot(x_ref[...], w_ref[...], preferred_element_type=jnp.float32),
            0.0)

    return pl.pallas_call(
        body,
        grid=(NP // RB,),
        in_specs=[pl.BlockSpec((RB, DF), lambda i: (i, 0)),
                  pl.BlockSpec((DF, HID), lambda i: (0, 0))],
        out_specs=pl.BlockSpec((RB, HID), lambda i: (i, 0)),
        out_shape=jax.ShapeDtypeStruct((NP, HID), jnp.float32),
    )(xp, w)


def _classify(rf, wc):
    RB = 512
    CD = rf.shape[1]

    def body(r_ref, w_ref, o_ref):
        logits = jnp.dot(r_ref[...], w_ref[...],
                         preferred_element_type=jnp.float32)
        m = jnp.max(logits, axis=1, keepdims=True)
        e = jnp.exp(logits - m)
        o_ref[...] = e / jnp.sum(e, axis=1, keepdims=True)

    return pl.pallas_call(
        body,
        grid=(NP // RB,),
        in_specs=[pl.BlockSpec((RB, CD), lambda i: (i, 0)),
                  pl.BlockSpec((CD, NCLS), lambda i: (0, 0))],
        out_specs=pl.BlockSpec((RB, NCLS), lambda i: (i, 0)),
        out_shape=jax.ShapeDtypeStruct((NP, NCLS), jnp.float32),
    )(rf, wc)


def kernel(x, edge_index, w_embed, w_classify):
    edge_flat = edge_index.astype(jnp.int32).reshape(2 * E)
    counts = _sc_build_adj(edge_flat).reshape(NP, NP)
    ahat, d = _prep(counts)

    dinv_row = jnp.where(d > 0, 1.0 / d, 0.0).reshape(1, NP)
    m2, deg2 = _two_hop(ahat, dinv_row)

    d1 = jnp.maximum(d - 1.0, 0.0)
    d1i = jnp.where(d1 > 0, lax.rsqrt(d1), 0.0)
    d2i = jnp.where(deg2 > 0, lax.rsqrt(deg2), 0.0)

    xp = jnp.pad(x, ((0, NP - N), (0, 0)))
    r0 = _embed(xp, w_embed)
    r1 = _spmm_round(ahat, m2, r0, d1i, d2i, d1i, d2i)
    r2 = _spmm_round(ahat, m2, r1, d1i, d2i, d1i, d2i)

    cat_dim = (2 ** 3 - 1) * HID
    rf = jnp.concatenate([r0, r1, r2, jnp.zeros((NP, 512 - cat_dim), jnp.float32)],
                         axis=1)
    wc = jnp.pad(w_classify, ((0, 512 - cat_dim), (0, 0)))
    out = _classify(rf, wc)
    return out[:N]
